# Initial kernel scaffold; baseline (speedup 1.0000x reference)
import jax
import jax.numpy as jnp
from jax.experimental import pallas as pl


def kernel(src, dst, t, msg, e_hist, t_hist, tw):
    raise NotImplementedError("write your pallas kernel here")


# speedup vs baseline: 13.6377x; 13.6377x over previous
import functools

import jax
import jax.numpy as jnp
from jax import lax
from jax.experimental import pallas as pl
from jax.experimental.pallas import tpu as pltpu
from jax.experimental.pallas import tpu_sc as plsc

L = 16
CHN = 16384
C2 = 512


def _lane_min(v):
    s = v[0]
    for i in range(1, L):
        s = jnp.minimum(s, v[i])
    return s


def _lane_sum(v):
    s = v[0]
    for i in range(1, L):
        s = s + v[i]
    return s


def _sc_body(B, H, NC, NS, is64,
             t_hbm, th_hbm, kh_hbm, kl_hbm, ehb_hbm, tw_hbm, out_hbm,
             tbuf, khbuf, klbuf, wbuf, accbuf, fbuf, s16, rdbuf, twbuf,
             sh_lo, sh_hi):
    c = lax.axis_index("c")
    s = lax.axis_index("s")
    w = c * NS + s
    kpw = B // (NC * NS)
    z16i = jnp.zeros((L,), jnp.int32)
    one16 = jnp.ones((L,), jnp.int32)
    i0 = jnp.int32(0)

    pltpu.sync_copy(t_hbm, tbuf)
    pltpu.sync_copy(tw_hbm, twbuf)

    def min_body(i, m):
        return jnp.minimum(m, tbuf[pl.ds(i * L, L)])

    mvec = lax.fori_loop(i0, jnp.int32(B // L), min_body,
                         jnp.full((L,), jnp.inf, jnp.float32))
    mmin = _lane_min(mvec)
    end_v = jnp.zeros((L,), jnp.float32) + mmin
    start_v = end_v - twbuf[...]


    NFULL = H // CHN
    TAIL = H - NFULL * CHN

    def count_chunk(n_vecs, acc):
        def body(i, a):
            x = tbuf[pl.ds(i * L, L)]
            cl, ch = a
            cl = cl + jnp.where(x < start_v, one16, z16i)
            ch = ch + jnp.where(x <= end_v, one16, z16i)
            return (cl, ch)
        return lax.fori_loop(i0, jnp.int32(n_vecs), body, acc)

    n_my_chunks = (NFULL - s + NS - 1) // NS

    def chunk1_body(cc, acc):
        ck = s + cc * NS
        pltpu.sync_copy(th_hbm.at[pl.ds(ck * CHN, CHN)], tbuf.at[pl.ds(0, CHN)])
        return count_chunk(CHN // L, acc)

    cl, ch = lax.fori_loop(i0, n_my_chunks, chunk1_body, (z16i, z16i))

    if TAIL > 0:
        pltpu.sync_copy(th_hbm.at[pl.ds(NFULL * CHN, TAIL)],
                        tbuf.at[pl.ds(0, TAIL)])
        tcl, tch = count_chunk(TAIL // L, (z16i, z16i))
        keep_tail = jnp.where(s == jnp.int32(NFULL % NS),
                              jnp.int32(1), jnp.int32(0))
        cl = cl + tcl * (z16i + keep_tail)
        ch = ch + tch * (z16i + keep_tail)

    s16[...] = z16i + _lane_sum(cl)
    pltpu.sync_copy(s16, sh_lo.at[pl.ds(s * L, L)])
    s16[...] = z16i + _lane_sum(ch)
    pltpu.sync_copy(s16, sh_hi.at[pl.ds(s * L, L)])
    plsc.subcore_barrier()

    pltpu.sync_copy(sh_lo, rdbuf)
    lo_v = z16i
    for r in range(NS):
        lo_v = lo_v + rdbuf[pl.ds(r * L, L)]
    pltpu.sync_copy(sh_hi, rdbuf)
    hi_v = z16i
    for r in range(NS):
        hi_v = hi_v + rdbuf[pl.ds(r * L, L)]
    lo_s = lo_v[0]
    hi_s = hi_v[0]

    koff = w * kpw
    pltpu.sync_copy(kh_hbm.at[pl.ds(koff, kpw)], khbuf)
    pltpu.sync_copy(kl_hbm.at[pl.ds(koff, kpw)], klbuf)
    for kv in range(kpw // L):
        accbuf[pl.ds(kv * L, L)] = z16i

    lo8 = (lo_s // jnp.int32(8)) * jnp.int32(8)
    nchunks = (hi_s - lo8 + jnp.int32(C2 - 1)) // jnp.int32(C2)

    def chunk2_body(k, _):
        base_k = lo8 + k * jnp.int32(C2)
        base_c = jnp.minimum(base_k, jnp.int32(H - C2))
        pltpu.sync_copy(ehb_hbm.at[pl.ds(base_c * 2, 2 * C2)], wbuf)
        jlo = jnp.maximum(lo_s, base_k) - base_c
        jhi = jnp.minimum(hi_s, base_k + jnp.int32(C2)) - base_c
        q0 = jlo // jnp.int32(8)
        q1 = (jhi + jnp.int32(7)) // jnp.int32(8)

        def q_body(q, __):
            pv = wbuf[pl.ds(q * L, L)]
            e0 = q * jnp.int32(8)
            for m in range(8):
                e = e0 + jnp.int32(m)
                valid = (e >= jlo) & (e < jhi)
                whs = jnp.where(valid, pv[2 * m + 1], jnp.int32(-2))
                wls = pv[2 * m]
                whv = z16i + whs
                wlv = z16i + wls
                for kv in range(kpw // L):
                    sl = pl.ds(kv * L, L)
                    eqh = jnp.where(khbuf[sl] == whv, one16, z16i)
                    eqb = jnp.where(klbuf[sl] == wlv, eqh, z16i)
                    accbuf[sl] = jnp.maximum(accbuf[sl], eqb)
            return i0

        lax.fori_loop(q0, q1, q_body, i0)
        return i0

    lax.fori_loop(i0, nchunks, chunk2_body, i0)

    sent_h = jnp.int32(-1) if is64 else jnp.int32(0)
    any_masked = jnp.where((jnp.int32(H) - (hi_s - lo_s)) > i0,
                           jnp.int32(1), jnp.int32(0))
    onef = jnp.ones((L,), jnp.float32)
    zerof = jnp.zeros((L,), jnp.float32)
    for kv in range(kpw // L):
        sl = pl.ds(kv * L, L)
        n1 = jnp.where(khbuf[sl] == (z16i + sent_h), one16, z16i)
        n2 = jnp.where(klbuf[sl] == (z16i - one16), n1, z16i)
        hit = jnp.maximum(accbuf[sl], n2 * (z16i + any_masked))
        fbuf[sl] = jnp.where(hit == z16i, zerof, onef)
    pltpu.sync_copy(fbuf, out_hbm.at[pl.ds(koff, kpw)])


def kernel(src, dst, t, msg, e_hist, t_hist, tw):
    del msg
    B = t.shape[0]
    H = t_hist.shape[0]
    info = plsc.get_sparse_core_info()
    NC, NS = info.num_cores, info.num_subcores
    assert B % (NC * NS * 8) == 0 and B % L == 0
    assert H % L == 0 and H >= C2 and CHN % L == 0 and C2 % 8 == 0

    is64 = e_hist.dtype == jnp.int64
    if is64:
        a = src.astype(jnp.int64)
        b = dst.astype(jnp.int64)
        keys = jnp.where(a >= b, a * a + a + b, a + b * b)
        kb = lax.bitcast_convert_type(keys, jnp.int32)
        kl, kh = kb[:, 0], kb[:, 1]
        ehb = lax.bitcast_convert_type(e_hist, jnp.int32).reshape(2 * H)
    else:
        a = src.astype(jnp.int32)
        b = dst.astype(jnp.int32)
        kl = jnp.where(a >= b, a * a + a + b, a + b * b)
        kh = jnp.zeros((B,), jnp.int32)
        ehb = jnp.stack([e_hist.astype(jnp.int32),
                         jnp.zeros((H,), jnp.int32)], axis=-1).reshape(2 * H)

    tw16 = jnp.full((L,), tw, jnp.float32)
    t_hist = t_hist.astype(jnp.float32)
    t = t.astype(jnp.float32)

    mesh = plsc.VectorSubcoreMesh(core_axis_name="c", subcore_axis_name="s")
    fn = pl.kernel(
        functools.partial(_sc_body, B, H, NC, NS, is64),
        out_type=jax.ShapeDtypeStruct((B,), jnp.float32),
        mesh=mesh,
        scratch_types=[
            pltpu.VMEM((B,), jnp.float32),
            pltpu.VMEM((B // (NC * NS),), jnp.int32),
            pltpu.VMEM((B // (NC * NS),), jnp.int32),
            pltpu.VMEM((2 * C2,), jnp.int32),
            pltpu.VMEM((B // (NC * NS),), jnp.int32),
            pltpu.VMEM((B // (NC * NS),), jnp.float32),
            pltpu.VMEM((L,), jnp.int32),
            pltpu.VMEM((NS * L,), jnp.int32),
            pltpu.VMEM((L,), jnp.float32),
            pltpu.VMEM_SHARED((NS * L,), jnp.int32),
            pltpu.VMEM_SHARED((NS * L,), jnp.int32),
        ],
    )
    return fn(t, t_hist, kh, kl, ehb, tw16)


# speedup vs baseline: 191.2631x; 14.0246x over previous
import functools

import jax
import jax.numpy as jnp
from jax import lax
from jax.experimental import pallas as pl
from jax.experimental.pallas import tpu as pltpu
from jax.experimental.pallas import tpu_sc as plsc

L = 16
CHN = 16384
C2 = 512


def _lane_min(v):
    s = v[0]
    for i in range(1, L):
        s = jnp.minimum(s, v[i])
    return s


def _lane_sum(v):
    s = v[0]
    for i in range(1, L):
        s = s + v[i]
    return s


def _sc_body(B, H, NC, NS, is64,
             t_hbm, th_hbm, kh_hbm, kl_hbm, ehl_hbm, ehh_hbm, tw_hbm, out_hbm,
             tbuf, khbuf, klbuf, wlbuf, whbuf, accbuf, fbuf, s16, rdbuf, twbuf,
             sh_lo, sh_hi):
    c = lax.axis_index("c")
    s = lax.axis_index("s")
    w = c * NS + s
    kpw = B // (NC * NS)
    z16i = jnp.zeros((L,), jnp.int32)
    one16 = jnp.ones((L,), jnp.int32)
    i0 = jnp.int32(0)

    pltpu.sync_copy(t_hbm, tbuf)
    pltpu.sync_copy(tw_hbm, twbuf)

    def min_body(i, m):
        return jnp.minimum(m, tbuf[pl.ds(i * L, L)])

    mvec = lax.fori_loop(i0, jnp.int32(B // L), min_body,
                         jnp.full((L,), jnp.inf, jnp.float32))
    mmin = _lane_min(mvec)
    end_v = jnp.zeros((L,), jnp.float32) + mmin
    start_v = end_v - twbuf[...]


    NFULL = H // CHN
    TAIL = H - NFULL * CHN

    def count_chunk(n_vecs, acc):
        def body(i, a):
            x = tbuf[pl.ds(i * L, L)]
            cl, ch = a
            cl = cl + jnp.where(x < start_v, one16, z16i)
            ch = ch + jnp.where(x <= end_v, one16, z16i)
            return (cl, ch)
        return lax.fori_loop(i0, jnp.int32(n_vecs), body, acc)

    n_my_chunks = (NFULL - s + NS - 1) // NS

    def chunk1_body(cc, acc):
        ck = s + cc * NS
        pltpu.sync_copy(th_hbm.at[pl.ds(ck * CHN, CHN)], tbuf.at[pl.ds(0, CHN)])
        return count_chunk(CHN // L, acc)

    cl, ch = lax.fori_loop(i0, n_my_chunks, chunk1_body, (z16i, z16i))

    if TAIL > 0:
        pltpu.sync_copy(th_hbm.at[pl.ds(NFULL * CHN, TAIL)],
                        tbuf.at[pl.ds(0, TAIL)])
        tcl, tch = count_chunk(TAIL // L, (z16i, z16i))
        keep_tail = jnp.where(s == jnp.int32(NFULL % NS),
                              jnp.int32(1), jnp.int32(0))
        cl = cl + tcl * (z16i + keep_tail)
        ch = ch + tch * (z16i + keep_tail)

    s16[...] = z16i + _lane_sum(cl)
    pltpu.sync_copy(s16, sh_lo.at[pl.ds(s * L, L)])
    s16[...] = z16i + _lane_sum(ch)
    pltpu.sync_copy(s16, sh_hi.at[pl.ds(s * L, L)])
    plsc.subcore_barrier()

    pltpu.sync_copy(sh_lo, rdbuf)
    lo_v = z16i
    for r in range(NS):
        lo_v = lo_v + rdbuf[pl.ds(r * L, L)]
    pltpu.sync_copy(sh_hi, rdbuf)
    hi_v = z16i
    for r in range(NS):
        hi_v = hi_v + rdbuf[pl.ds(r * L, L)]
    lo_s = lo_v[0]
    hi_s = hi_v[0]

    koff = w * kpw
    pltpu.sync_copy(kh_hbm.at[pl.ds(koff, kpw)], khbuf)
    pltpu.sync_copy(kl_hbm.at[pl.ds(koff, kpw)], klbuf)
    for kv in range(kpw // L):
        accbuf[pl.ds(kv * L, L)] = z16i

    lo8 = (lo_s // jnp.int32(8)) * jnp.int32(8)
    nchunks = (hi_s - lo8 + jnp.int32(C2 - 1)) // jnp.int32(C2)

    def chunk2_body(k, _):
        base_k = lo8 + k * jnp.int32(C2)
        base_c = jnp.minimum(base_k, jnp.int32(H - C2))
        pltpu.sync_copy(ehl_hbm.at[pl.ds(base_c, C2)], wlbuf)
        pltpu.sync_copy(ehh_hbm.at[pl.ds(base_c, C2)], whbuf)
        jlo = jnp.maximum(lo_s, base_k) - base_c
        jhi = jnp.minimum(hi_s, base_k + jnp.int32(C2)) - base_c
        q0 = jlo // jnp.int32(L)
        q1 = (jhi + jnp.int32(L - 1)) // jnp.int32(L)

        def q_body(q, __):
            vl = wlbuf[pl.ds(q * L, L)]
            vh = whbuf[pl.ds(q * L, L)]
            e0 = q * jnp.int32(L)
            for m in range(L):
                e = e0 + jnp.int32(m)
                valid = (e >= jlo) & (e < jhi)
                whs = jnp.where(valid, vh[m], jnp.int32(-2))
                whv = z16i + whs
                wlv = z16i + vl[m]
                for kv in range(kpw // L):
                    sl = pl.ds(kv * L, L)
                    eqh = jnp.where(khbuf[sl] == whv, one16, z16i)
                    eqb = jnp.where(klbuf[sl] == wlv, eqh, z16i)
                    accbuf[sl] = jnp.maximum(accbuf[sl], eqb)
            return i0

        lax.fori_loop(q0, q1, q_body, i0)
        return i0

    lax.fori_loop(i0, nchunks, chunk2_body, i0)

    sent_h = jnp.int32(-1) if is64 else jnp.int32(0)
    any_masked = jnp.where((jnp.int32(H) - (hi_s - lo_s)) > i0,
                           jnp.int32(1), jnp.int32(0))
    onef = jnp.ones((L,), jnp.float32)
    zerof = jnp.zeros((L,), jnp.float32)
    for kv in range(kpw // L):
        sl = pl.ds(kv * L, L)
        n1 = jnp.where(khbuf[sl] == (z16i + sent_h), one16, z16i)
        n2 = jnp.where(klbuf[sl] == (z16i - one16), n1, z16i)
        hit = jnp.maximum(accbuf[sl], n2 * (z16i + any_masked))
        fbuf[sl] = jnp.where(hit == z16i, zerof, onef)
    pltpu.sync_copy(fbuf, out_hbm.at[pl.ds(koff, kpw)])


def kernel(src, dst, t, msg, e_hist, t_hist, tw):
    del msg
    B = t.shape[0]
    H = t_hist.shape[0]
    info = plsc.get_sparse_core_info()
    NC, NS = info.num_cores, info.num_subcores
    assert B % (NC * NS * 8) == 0 and B % L == 0
    assert H % L == 0 and H >= C2 and CHN % L == 0 and C2 % 8 == 0

    is64 = e_hist.dtype == jnp.int64
    if is64:
        a = src.astype(jnp.int64)
        b = dst.astype(jnp.int64)
        keys = jnp.where(a >= b, a * a + a + b, a + b * b)
        kl = lax.convert_element_type(keys, jnp.int32)
        kh = lax.convert_element_type(
            lax.shift_right_arithmetic(keys, jnp.int64(32)), jnp.int32)
        ehl = lax.convert_element_type(e_hist, jnp.int32)
        ehh = lax.convert_element_type(
            lax.shift_right_arithmetic(e_hist, jnp.int64(32)), jnp.int32)
    else:
        a = src.astype(jnp.int32)
        b = dst.astype(jnp.int32)
        kl = jnp.where(a >= b, a * a + a + b, a + b * b)
        kh = jnp.zeros((B,), jnp.int32)
        ehl = e_hist.astype(jnp.int32)
        ehh = jnp.zeros((H,), jnp.int32)

    tw16 = jnp.full((L,), tw, jnp.float32)
    t_hist = t_hist.astype(jnp.float32)
    t = t.astype(jnp.float32)

    mesh = plsc.VectorSubcoreMesh(core_axis_name="c", subcore_axis_name="s")
    fn = pl.kernel(
        functools.partial(_sc_body, B, H, NC, NS, is64),
        out_type=jax.ShapeDtypeStruct((B,), jnp.float32),
        mesh=mesh,
        scratch_types=[
            pltpu.VMEM((B,), jnp.float32),
            pltpu.VMEM((B // (NC * NS),), jnp.int32),
            pltpu.VMEM((B // (NC * NS),), jnp.int32),
            pltpu.VMEM((C2,), jnp.int32),
            pltpu.VMEM((C2,), jnp.int32),
            pltpu.VMEM((B // (NC * NS),), jnp.int32),
            pltpu.VMEM((B // (NC * NS),), jnp.float32),
            pltpu.VMEM((L,), jnp.int32),
            pltpu.VMEM((NS * L,), jnp.int32),
            pltpu.VMEM((L,), jnp.float32),
            pltpu.VMEM_SHARED((NS * L,), jnp.int32),
            pltpu.VMEM_SHARED((NS * L,), jnp.int32),
        ],
    )
    return fn(t, t_hist, kh, kl, ehl, ehh, tw16)


# speedup vs baseline: 230.0026x; 1.2025x over previous
import functools

import jax
import jax.numpy as jnp
from jax import lax
from jax.experimental import pallas as pl
from jax.experimental.pallas import tpu as pltpu
from jax.experimental.pallas import tpu_sc as plsc

L = 16
SAMP = 1024
C2 = 512


def _lane_min(v):
    s = v[0]
    for i in range(1, L):
        s = jnp.minimum(s, v[i])
    return s


def _lane_sum(v):
    s = v[0]
    for i in range(1, L):
        s = s + v[i]
    return s


def _sc_body(B, H, NC, NS, SPAD, is64,
             t_hbm, th_hbm, kh_hbm, kl_hbm, ehl_hbm, ehh_hbm, tw_hbm,
             tsamp_hbm, out_hbm,
             tbuf, khbuf, klbuf, wlbuf, whbuf, accbuf, fbuf, twbuf,
             sbuf, bbuf):
    c = lax.axis_index("c")
    s = lax.axis_index("s")
    w = c * NS + s
    kpw = B // (NC * NS)
    z16i = jnp.zeros((L,), jnp.int32)
    one16 = jnp.ones((L,), jnp.int32)
    i0 = jnp.int32(0)

    pltpu.sync_copy(t_hbm, tbuf)
    pltpu.sync_copy(tw_hbm, twbuf)

    def min_body(i, m):
        base = i * jnp.int32(4 * L)
        for u in range(4):
            m = jnp.minimum(m, tbuf[pl.ds(base + jnp.int32(u * L), L)])
        return m

    mvec = lax.fori_loop(i0, jnp.int32(B // (4 * L)), min_body,
                         jnp.full((L,), jnp.inf, jnp.float32))
    mmin = _lane_min(mvec)
    end_v = jnp.zeros((L,), jnp.float32) + mmin
    start_v = end_v - twbuf[...]

    pltpu.sync_copy(tsamp_hbm, sbuf)
    csl = z16i
    csh = z16i
    for q in range(SPAD // L):
        x = sbuf[pl.ds(q * L, L)]
        csl = csl + jnp.where(x < start_v, one16, z16i)
        csh = csh + jnp.where(x <= end_v, one16, z16i)
    c_lo = _lane_sum(csl)
    c_hi = _lane_sum(csh)

    def band_bound(c_s, is_hi):
        bsc = jnp.where(c_s > i0,
                        jnp.minimum((c_s - jnp.int32(1)) * jnp.int32(SAMP),
                                    jnp.int32(H - SAMP)),
                        i0)
        pltpu.sync_copy(th_hbm.at[pl.ds(bsc, SAMP)], bbuf)
        acc = z16i
        for q in range(SAMP // L):
            x = bbuf[pl.ds(q * L, L)]
            if is_hi:
                acc = acc + jnp.where(x <= end_v, one16, z16i)
            else:
                acc = acc + jnp.where(x < start_v, one16, z16i)
        cnt = _lane_sum(acc)
        return jnp.where(c_s > i0, bsc + cnt, i0)

    lo_s = band_bound(c_lo, False)
    hi_s = band_bound(c_hi, True)

    koff = w * kpw
    pltpu.sync_copy(kh_hbm.at[pl.ds(koff, kpw)], khbuf)
    pltpu.sync_copy(kl_hbm.at[pl.ds(koff, kpw)], klbuf)
    for kv in range(kpw // L):
        accbuf[pl.ds(kv * L, L)] = z16i

    lo8 = (lo_s // jnp.int32(8)) * jnp.int32(8)
    nchunks = (hi_s - lo8 + jnp.int32(C2 - 1)) // jnp.int32(C2)

    def chunk2_body(k, _):
        base_k = lo8 + k * jnp.int32(C2)
        base_c = jnp.minimum(base_k, jnp.int32(H - C2))
        pltpu.sync_copy(ehl_hbm.at[pl.ds(base_c, C2)], wlbuf)
        pltpu.sync_copy(ehh_hbm.at[pl.ds(base_c, C2)], whbuf)
        jlo = jnp.maximum(lo_s, base_k) - base_c
        jhi = jnp.minimum(hi_s, base_k + jnp.int32(C2)) - base_c
        q0 = jlo // jnp.int32(L)
        q1 = (jhi + jnp.int32(L - 1)) // jnp.int32(L)

        def q_body(q, __):
            vl = wlbuf[pl.ds(q * L, L)]
            vh = whbuf[pl.ds(q * L, L)]
            e0 = q * jnp.int32(L)
            for m in range(L):
                e = e0 + jnp.int32(m)
                valid = (e >= jlo) & (e < jhi)
                whs = jnp.where(valid, vh[m], jnp.int32(-2))
                whv = z16i + whs
                wlv = z16i + vl[m]
                for kv in range(kpw // L):
                    sl = pl.ds(kv * L, L)
                    eqh = jnp.where(khbuf[sl] == whv, one16, z16i)
                    eqb = jnp.where(klbuf[sl] == wlv, eqh, z16i)
                    accbuf[sl] = jnp.maximum(accbuf[sl], eqb)
            return i0

        lax.fori_loop(q0, q1, q_body, i0)
        return i0

    lax.fori_loop(i0, nchunks, chunk2_body, i0)

    sent_h = jnp.int32(-1) if is64 else jnp.int32(0)
    any_masked = jnp.where((jnp.int32(H) - (hi_s - lo_s)) > i0,
                           jnp.int32(1), jnp.int32(0))
    onef = jnp.ones((L,), jnp.float32)
    zerof = jnp.zeros((L,), jnp.float32)
    for kv in range(kpw // L):
        sl = pl.ds(kv * L, L)
        n1 = jnp.where(khbuf[sl] == (z16i + sent_h), one16, z16i)
        n2 = jnp.where(klbuf[sl] == (z16i - one16), n1, z16i)
        hit = jnp.maximum(accbuf[sl], n2 * (z16i + any_masked))
        fbuf[sl] = jnp.where(hit == z16i, zerof, onef)
    pltpu.sync_copy(fbuf, out_hbm.at[pl.ds(koff, kpw)])


def kernel(src, dst, t, msg, e_hist, t_hist, tw):
    del msg
    B = t.shape[0]
    H = t_hist.shape[0]
    info = plsc.get_sparse_core_info()
    NC, NS = info.num_cores, info.num_subcores
    assert B % (NC * NS * 8) == 0 and B % L == 0
    assert H % L == 0 and H >= C2 and C2 % 8 == 0

    NSAMP = -(-H // SAMP)
    SPAD = -(-NSAMP // L) * L
    assert H % 8 == 0 and (H - SAMP) % 8 == 0 and H >= SAMP

    is64 = e_hist.dtype == jnp.int64
    if is64:
        a = src.astype(jnp.int64)
        b = dst.astype(jnp.int64)
        keys = jnp.where(a >= b, a * a + a + b, a + b * b)
        kl = lax.convert_element_type(keys, jnp.int32)
        kh = lax.convert_element_type(
            lax.shift_right_arithmetic(keys, jnp.int64(32)), jnp.int32)
        ehl = lax.convert_element_type(e_hist, jnp.int32)
        ehh = lax.convert_element_type(
            lax.shift_right_arithmetic(e_hist, jnp.int64(32)), jnp.int32)
    else:
        a = src.astype(jnp.int32)
        b = dst.astype(jnp.int32)
        kl = jnp.where(a >= b, a * a + a + b, a + b * b)
        kh = jnp.zeros((B,), jnp.int32)
        ehl = e_hist.astype(jnp.int32)
        ehh = jnp.zeros((H,), jnp.int32)

    tw16 = jnp.full((L,), tw, jnp.float32)
    t_hist = t_hist.astype(jnp.float32)
    t = t.astype(jnp.float32)
    tsamp = jnp.concatenate(
        [t_hist[::SAMP],
         jnp.full((SPAD - NSAMP,), jnp.inf, jnp.float32)])

    mesh = plsc.VectorSubcoreMesh(core_axis_name="c", subcore_axis_name="s")
    fn = pl.kernel(
        functools.partial(_sc_body, B, H, NC, NS, SPAD, is64),
        out_type=jax.ShapeDtypeStruct((B,), jnp.float32),
        mesh=mesh,
        scratch_types=[
            pltpu.VMEM((B,), jnp.float32),
            pltpu.VMEM((B // (NC * NS),), jnp.int32),
            pltpu.VMEM((B // (NC * NS),), jnp.int32),
            pltpu.VMEM((C2,), jnp.int32),
            pltpu.VMEM((C2,), jnp.int32),
            pltpu.VMEM((B // (NC * NS),), jnp.int32),
            pltpu.VMEM((B // (NC * NS),), jnp.float32),
            pltpu.VMEM((L,), jnp.float32),
            pltpu.VMEM((SPAD,), jnp.float32),
            pltpu.VMEM((SAMP,), jnp.float32),
        ],
    )
    return fn(t, t_hist, kh, kl, ehl, ehh, tw16, tsamp)
